# Initial kernel scaffold; baseline (speedup 1.0000x reference)
import jax
import jax.numpy as jnp
from jax.experimental import pallas as pl


def kernel(x, edge_index, W1, a_src1, a_dst1, b1, W2, a_src2, a_dst2, b2):
    raise NotImplementedError("write your pallas kernel here")


# speedup vs baseline: 19.0468x; 19.0468x over previous
import functools

import jax
import jax.numpy as jnp
from jax import lax
from jax.experimental import pallas as pl
from jax.experimental.pallas import tpu as pltpu
from jax.experimental.pallas import tpu_sc as plsc

NC = 2
NS = 16
K = 80
HPAD = 16
CW = 64


def _tc1_body(x_ref, w_ref, asr_ref, adr_ref, ha_ref, hb_ref, as_ref, ad_ref):
    h = jnp.dot(x_ref[...], w_ref[...], preferred_element_type=jnp.float32)
    ha_ref[...] = h[:, :128]
    hb_ref[...] = h[:, 128:]
    bm = h.shape[0]
    hr = h.reshape(bm, 4, 64)
    as_ref[...] = jnp.sum(hr * asr_ref[...][None], axis=-1)
    ad_ref[...] = jnp.sum(hr * adr_ref[...][None], axis=-1)


@functools.lru_cache(maxsize=None)
def _make_tc1(n, bm):
    grid = (n // bm,)
    return pl.pallas_call(
        _tc1_body,
        grid=grid,
        in_specs=[
            pl.BlockSpec((bm, 128), lambda i: (i, 0)),
            pl.BlockSpec((128, 256), lambda i: (0, 0)),
            pl.BlockSpec((4, 64), lambda i: (0, 0)),
            pl.BlockSpec((4, 64), lambda i: (0, 0)),
        ],
        out_specs=[
            pl.BlockSpec((bm, 128), lambda i: (i, 0)),
            pl.BlockSpec((bm, 128), lambda i: (i, 0)),
            pl.BlockSpec((bm, 4), lambda i: (i, 0)),
            pl.BlockSpec((bm, 4), lambda i: (i, 0)),
        ],
        out_shape=[
            jax.ShapeDtypeStruct((n, 128), jnp.float32),
            jax.ShapeDtypeStruct((n, 128), jnp.float32),
            jax.ShapeDtypeStruct((n, 4), jnp.float32),
            jax.ShapeDtypeStruct((n, 4), jnp.float32),
        ],
    )


def _tc2_body(accA_ref, denA_ref, accB_ref, denB_ref, b1_ref, w2_ref,
              as2_ref, ad2_ref, h2_ref, sd2_ref):
    eps = 1e-16
    hA0 = accA_ref[0] / (denA_ref[0][:, 0:1] + eps)
    hA1 = accA_ref[1] / (denA_ref[1][:, 0:1] + eps)
    hB0 = accB_ref[0] / (denB_ref[0][:, 0:1] + eps)
    hB1 = accB_ref[1] / (denB_ref[1][:, 0:1] + eps)
    h1 = jnp.concatenate([hA0, hA1, hB0, hB1], axis=1) + b1_ref[...][None]
    h1 = jnp.where(h1 > 0, h1, jnp.exp(jnp.minimum(h1, 0.0)) - 1.0)
    h2 = jnp.dot(h1, w2_ref[...], preferred_element_type=jnp.float32)
    h2_ref[...] = h2
    s2 = jnp.sum(h2 * as2_ref[...][0][None], axis=-1)
    d2 = jnp.sum(h2 * ad2_ref[...][0][None], axis=-1)
    sd2_ref[...] = jnp.stack([s2, d2], axis=1)


@functools.lru_cache(maxsize=None)
def _make_tc2(n, bm):
    grid = (n // bm,)
    return pl.pallas_call(
        _tc2_body,
        grid=grid,
        in_specs=[
            pl.BlockSpec((NC, bm, CW), lambda i: (0, i, 0)),
            pl.BlockSpec((NC, bm, HPAD), lambda i: (0, i, 0)),
            pl.BlockSpec((NC, bm, CW), lambda i: (0, i, 0)),
            pl.BlockSpec((NC, bm, HPAD), lambda i: (0, i, 0)),
            pl.BlockSpec((256,), lambda i: (0,)),
            pl.BlockSpec((256, 128), lambda i: (0, 0)),
            pl.BlockSpec((1, 128), lambda i: (0, 0)),
            pl.BlockSpec((1, 128), lambda i: (0, 0)),
        ],
        out_specs=[
            pl.BlockSpec((bm, 128), lambda i: (i, 0)),
            pl.BlockSpec((bm, 2), lambda i: (i, 0)),
        ],
        out_shape=[
            jax.ShapeDtypeStruct((n, 128), jnp.float32),
            jax.ShapeDtypeStruct((n, 2), jnp.float32),
        ],
    )


def _tc3_body(acc_ref, den_ref, b2_ref, out_ref):
    a = jnp.concatenate([acc_ref[0], acc_ref[1]], axis=1)
    d = den_ref[0][:, 0:1]
    out_ref[...] = a / (d + 1e-16) + b2_ref[...][None]


@functools.lru_cache(maxsize=None)
def _make_tc3(n, bm):
    grid = (n // bm,)
    return pl.pallas_call(
        _tc3_body,
        grid=grid,
        in_specs=[
            pl.BlockSpec((NC, bm, CW), lambda i: (0, i, 0)),
            pl.BlockSpec((NC, bm, HPAD), lambda i: (0, i, 0)),
            pl.BlockSpec((128,), lambda i: (0,)),
        ],
        out_specs=pl.BlockSpec((bm, 128), lambda i: (i, 0)),
        out_shape=jax.ShapeDtypeStruct((n, 128), jnp.float32),
    )


@functools.lru_cache(maxsize=None)
def _make_sc_edge(n, e, h_sub):
    e_per_t = e // NS
    n_chunks = e_per_t // K
    RA = (n // (NS * 8)) * 8
    RLAST = n - (NS - 1) * RA

    def _zchunks(nrows):
        out, off = [], 0
        while off < nrows:
            sz = min(K, nrows - off)
            out.append((off, sz))
            off += sz
        return out

    mesh = plsc.VectorSubcoreMesh(core_axis_name="c", subcore_axis_name="s")

    def body(h_hbm, asrc_hbm, adst_hbm, src_hbm, dst_hbm,
             acc_out, den_out,
             srcv, dstv, rows, srcv2, dstv2, rows2, sflat, sbufw, asv, adv,
             accs, dens, sem, sem2):
        c = lax.axis_index("c")
        s = lax.axis_index("s")
        row0 = s * RA
        iota = lax.iota(jnp.int32, 16)

        zero16 = jnp.zeros((16,), jnp.float32)

        def zrow(i, carry):
            for j in range(CW // 16):
                rows[i, pl.ds(j * 16, 16)] = zero16
            sbufw[i, pl.ds(0, 16)] = zero16
            return carry
        lax.fori_loop(0, K, zrow, 0)

        def zero_rows(nrows):
            for (zo, zs) in _zchunks(nrows):
                pltpu.sync_copy(rows.at[pl.ds(0, zs)],
                                accs.at[pl.ds(row0 + zo, zs)])
                pltpu.sync_copy(sbufw.at[pl.ds(0, zs)],
                                dens.at[pl.ds(row0 + zo, zs)])
        pl.when(s < NS - 1)(lambda: zero_rows(RA))
        pl.when(s == NS - 1)(lambda: zero_rows(RLAST))
        plsc.subcore_barrier()

        pltpu.sync_copy(asrc_hbm, asv)
        pltpu.sync_copy(adst_hbm, adv)

        hoff = c if h_sub == 2 else 0
        lane0 = iota == 0

        def load_idx(ci, sv, dv):
            base = s * e_per_t + ci * K
            pltpu.sync_copy(src_hbm.at[pl.ds(base, K)], sv)
            pltpu.sync_copy(dst_hbm.at[pl.ds(base, K)], dv)

        def compute_scatter(rws, sv, dv):
            for j in range(K // 16):
                s16 = sv[pl.ds(j * 16, 16)]
                d16 = dv[pl.ds(j * 16, 16)]
                av = plsc.load_gather(asv, [s16 * h_sub + hoff])
                bv = plsc.load_gather(adv, [d16 * h_sub + hoff])
                ev = av + bv
                ev = jnp.maximum(ev, 0.2 * ev)
                sflat[pl.ds(j * 16, 16)] = jnp.exp(ev)

            def mul(eidx, carry2):
                sp = plsc.load_gather(sflat, [jnp.full((16,), eidx, jnp.int32)])
                sbufw[eidx, pl.ds(0, 16)] = jnp.where(lane0, sp, 0.0)
                for jj in range(CW // 16):
                    colo = jj * 16
                    rws[eidx, pl.ds(colo, 16)] = (
                        rws[eidx, pl.ds(colo, 16)] * sp)
                return carry2
            lax.fori_loop(0, K, mul, 0)
            pltpu.sync_copy(sbufw, dens.at[dv], add=True)
            pltpu.sync_copy(rws, accs.at[dv], add=True)

        load_idx(0, srcv, dstv)
        pltpu.async_copy(h_hbm.at[c].at[srcv], rows, sem)

        def pair(i, carry):
            load_idx(2 * i + 1, srcv2, dstv2)
            pltpu.async_copy(h_hbm.at[c].at[srcv2], rows2, sem2)
            pltpu.make_async_copy(h_hbm.at[c].at[srcv], rows, sem).wait()
            compute_scatter(rows, srcv, dstv)

            def refill():
                load_idx(2 * i + 2, srcv, dstv)
                pltpu.async_copy(h_hbm.at[c].at[srcv], rows, sem)
            pl.when(i < n_chunks // 2 - 1)(refill)
            pltpu.make_async_copy(h_hbm.at[c].at[srcv2], rows2, sem2).wait()
            compute_scatter(rows2, srcv2, dstv2)
            return carry
        lax.fori_loop(0, n_chunks // 2, pair, 0)

        plsc.subcore_barrier()

        def publish(nrows):
            pltpu.sync_copy(accs.at[pl.ds(row0, nrows)],
                            acc_out.at[c, pl.ds(row0, nrows)])
            pltpu.sync_copy(dens.at[pl.ds(row0, nrows)],
                            den_out.at[c, pl.ds(row0, nrows)])
        pl.when(s < NS - 1)(lambda: publish(RA))
        pl.when(s == NS - 1)(lambda: publish(RLAST))

    return pl.kernel(
        body,
        out_type=(jax.ShapeDtypeStruct((NC, n, CW), jnp.float32),
                  jax.ShapeDtypeStruct((NC, n, HPAD), jnp.float32)),
        mesh=mesh,
        compiler_params=pltpu.CompilerParams(
            needs_layout_passes=False, use_tc_tiling_on_sc=False),
        scratch_types=[
            pltpu.VMEM((K,), jnp.int32),
            pltpu.VMEM((K,), jnp.int32),
            pltpu.VMEM((K, CW), jnp.float32),
            pltpu.VMEM((K,), jnp.int32),
            pltpu.VMEM((K,), jnp.int32),
            pltpu.VMEM((K, CW), jnp.float32),
            pltpu.VMEM((K,), jnp.float32),
            pltpu.VMEM((K, HPAD), jnp.float32),
            pltpu.VMEM((n * h_sub,), jnp.float32),
            pltpu.VMEM((n * h_sub,), jnp.float32),
            pltpu.VMEM_SHARED((n, CW), jnp.float32),
            pltpu.VMEM_SHARED((n, HPAD), jnp.float32),
            pltpu.SemaphoreType.DMA,
            pltpu.SemaphoreType.DMA,
        ],
    )


def kernel(x, edge_index, W1, a_src1, a_dst1, b1, W2, a_src2, a_dst2, b2):
    n = x.shape[0]
    e = edge_index.shape[1]
    bm = 2000
    src = edge_index[0]
    dst = edge_index[1]

    ha, hb, asrc1, adst1 = _make_tc1(n, bm)(x, W1, a_src1, a_dst1)
    h3A = jnp.stack([ha[:, 0:CW], ha[:, CW:2 * CW]])
    h3B = jnp.stack([hb[:, 0:CW], hb[:, CW:2 * CW]])
    asA = asrc1[:, 0:2].reshape(-1)
    adA = adst1[:, 0:2].reshape(-1)
    asB = asrc1[:, 2:4].reshape(-1)
    adB = adst1[:, 2:4].reshape(-1)

    sc2 = _make_sc_edge(n, e, 2)
    accA, denA = sc2(h3A, asA, adA, src, dst)
    accB, denB = sc2(h3B, asB, adB, src, dst)

    h2, sd2 = _make_tc2(n, bm)(accA, denA, accB, denB, b1, W2,
                               a_src2, a_dst2)
    h3C = jnp.stack([h2[:, 0:CW], h2[:, CW:2 * CW]])
    as2 = sd2[:, 0]
    ad2 = sd2[:, 1]

    sc1 = _make_sc_edge(n, e, 1)
    acc2, den2 = sc1(h3C, as2, ad2, src, dst)

    return _make_tc3(n, bm)(acc2, den2, b2)
